# baseline (speedup 1.0000x reference)
import functools

import jax
import jax.numpy as jnp
from jax import lax
from jax.experimental import pallas as pl
from jax.experimental.pallas import tpu as pltpu
from jax.experimental.pallas import tpu_sc as plsc

_N = 10000
_NPAD = 10240
_K = 16
_LAT = 20
_LP = 32
_MU = 10.0
_BATCH = 16
_NCL = 50
_NB = 20
_JB = _NPAD // _NB
_KL = _K * _LP

_NW = 32
_ROWS = _NPAD * _K
_CHUNK = 640
_NBUF = 4


def _stage_a_body(x_ref, w1_ref, b1_ref, w2_ref, b2_ref, h0w_ref, h0b_ref,
                  h1w_ref, h1b_ref, h2w_ref, h2b_ref, bv_ref, a_ref, e_ref):
    f32 = jnp.float32
    hi = None
    pre = lax.dot_general(w1_ref[...], x_ref[...], (((1,), (1,)), ((), ())),
                          precision=hi,
                          preferred_element_type=jnp.float32
                          ) + b1_ref[...]
    s = jax.nn.sigmoid(pre)
    enc_t = lax.dot_general(w2_ref[...], s, (((1,), (0,)), ((), ())),
                            precision=hi) + b2_ref[...]
    h = lax.dot_general(h0w_ref[...], enc_t, (((1,), (0,)), ((), ())),
                        precision=hi) + h0b_ref[...]
    h = h * jax.nn.sigmoid(h)
    h = lax.dot_general(h1w_ref[...], h, (((1,), (0,)), ((), ())),
                        precision=hi) + h1b_ref[...]
    h = h * jax.nn.sigmoid(h)
    h = lax.dot_general(h2w_ref[...], h, (((1,), (0,)), ((), ())),
                        precision=hi) + h2b_ref[...]
    hot = jax.nn.sigmoid(0.01 * h)
    logu = jnp.log(1.0 - 0.5 * hot)
    c0 = (_MU * bv_ref[0, 0]) ** -2
    lvec = lax.broadcasted_iota(jnp.int32, (_NCL, _LP), 1).astype(f32)
    lmask = lvec < float(_LAT)
    for i in range(_BATCH):
        li = jnp.broadcast_to(logu[:, i:i + 1], (_NCL, _LP))
        a_ref[i] = jnp.where(lmask, c0 * jnp.exp(-2.0 * lvec * li), 0.0)
    e = jnp.transpose(enc_t)
    e_ref[...] = jnp.concatenate(
        [e, jnp.zeros((_BATCH, _LP - _LAT), f32)], axis=1)


def _stage_a(x, enc1_w, enc1_b, enc2_w, enc2_b, h0_w, h0_b, h1_w, h1_b,
             h2_w, h2_b, b_scalar):
    f32 = jnp.float32
    out_shape = (jax.ShapeDtypeStruct((_BATCH, _NCL, _LP), f32),
                 jax.ShapeDtypeStruct((_BATCH, _LP), f32))
    bf16 = jnp.bfloat16
    return pl.pallas_call(_stage_a_body, out_shape=out_shape)(
        x.astype(bf16), enc1_w.astype(bf16),
        enc1_b.reshape(-1, 1), enc2_w, enc2_b.reshape(-1, 1),
        h0_w, h0_b.reshape(-1, 1), h1_w, h1_b.reshape(-1, 1),
        h2_w, h2_b.reshape(-1, 1), b_scalar.reshape(1, 1))


def _gather(table, idx):
    n_rows = idx.shape[0]
    rpw = n_rows // _NW
    nch = rpw // _CHUNK
    mesh = plsc.VectorSubcoreMesh(core_axis_name="c", subcore_axis_name="s")

    @functools.partial(
        pl.kernel, mesh=mesh,
        compiler_params=pltpu.CompilerParams(use_tc_tiling_on_sc=False),
        out_type=jax.ShapeDtypeStruct((n_rows, _LP), jnp.bfloat16),
        scratch_types=([pltpu.VMEM((rpw,), jnp.int32)]
                       + [pltpu.VMEM((_CHUNK, _LP), jnp.bfloat16)] * _NBUF
                       + [pltpu.SemaphoreType.DMA] * (2 * _NBUF)),
    )
    def gk(table_hbm, idx_hbm, out_hbm, idx_v, *bufs_sems):
        rows = bufs_sems[:_NBUF]
        gsem = bufs_sems[_NBUF:2 * _NBUF]
        osem = bufs_sems[2 * _NBUF:]
        wid = lax.axis_index("s") * 2 + lax.axis_index("c")
        base = wid * rpw
        pltpu.sync_copy(idx_hbm.at[pl.ds(base, rpw)], idx_v)
        gh = {}
        sh = {}
        for c in range(nch + 1):
            b = c % _NBUF
            if c < nch:
                if c >= _NBUF:
                    sh[c - _NBUF].wait()
                gh[c] = pltpu.async_copy(
                    table_hbm.at[idx_v.at[pl.ds(c * _CHUNK, _CHUNK)]],
                    rows[b], gsem[b])
            d = c - 1
            if 0 <= d < nch:
                gh[d].wait()
                sh[d] = pltpu.async_copy(
                    rows[d % _NBUF],
                    out_hbm.at[pl.ds(base + d * _CHUNK, _CHUNK)],
                    osem[d % _NBUF])
        for d in range(max(0, nch - _NBUF), nch):
            sh[d].wait()

    return gk(table, idx)


def _main_body(nd_ref, g_ref, lab_ref, a_ref, e_ref, bias_ref, out_ref):
    f32 = jnp.float32
    bf16 = jnp.bfloat16
    nd = nd_ref[...]
    kt = ((lax.broadcasted_iota(jnp.int32, (_K, _KL), 1) // _LP) ==
          lax.broadcasted_iota(jnp.int32, (_K, _KL), 0)).astype(bf16)
    d2 = jnp.dot((nd * nd).astype(bf16), kt,
                 preferred_element_type=f32).astype(bf16)
    g = g_ref[...]
    onehot = (jnp.broadcast_to(lab_ref[...], (_JB, _NCL)) ==
              lax.broadcasted_iota(jnp.int32, (_JB, _NCL), 1).astype(f32)
              ).astype(bf16)
    seg = ((lax.broadcasted_iota(jnp.int32, (_KL, _LP), 0) % _LP) ==
           lax.broadcasted_iota(jnp.int32, (_KL, _LP), 1)).astype(bf16)
    a_bf = a_ref[...].astype(bf16)
    cols = []
    for i in range(_BATCH):
        a32 = jnp.dot(onehot, a_bf[i],
                      preferred_element_type=f32).astype(bf16)
        at = jnp.concatenate([a32] * _K, axis=1)
        r = jnp.maximum(1.0 - d2 * at, 0.0)
        rg = r * g
        zq = jnp.dot(jnp.concatenate([r, rg], axis=0), seg,
                     preferred_element_type=f32)
        z = zq[:_JB]
        q = zq[_JB:]
        ei = e_ref[i:i + 1, :]
        cols.append(jnp.sum(q * (ei / z), axis=1, keepdims=True))
    out_ref[...] = jnp.concatenate(cols, axis=1) + bias_ref[...]


def _main(nd_t, g2, lab_f, a_tab, e_pad, bias2, blk0):
    f32 = jnp.float32
    nblk = g2.shape[0] // _JB
    return pl.pallas_call(
        _main_body,
        grid=(nblk,),
        in_specs=[
            pl.BlockSpec((_JB, _K), lambda b: (b + blk0, 0)),
            pl.BlockSpec((_JB, _KL), lambda b: (b, 0)),
            pl.BlockSpec((_JB, 1), lambda b: (b + blk0, 0)),
            pl.BlockSpec((_BATCH, _NCL, _LP), lambda b: (0, 0, 0)),
            pl.BlockSpec((_BATCH, _LP), lambda b: (0, 0)),
            pl.BlockSpec((_JB, 1), lambda b: (b + blk0, 0)),
        ],
        out_specs=pl.BlockSpec((_JB, _BATCH), lambda b: (b, 0)),
        out_shape=jax.ShapeDtypeStruct((g2.shape[0], _BATCH), f32),
    )(nd_t, g2, lab_f, a_tab, e_pad, bias2)


def kernel(x, enc1_w, enc1_b, enc2_w, enc2_b, dec_w, dec_b, h0_w, h0_b,
           h1_w, h1_b, h2_w, h2_b, B, neighbour_id, neighbour_distance,
           clustering_labels):
    f32 = jnp.float32
    a_tab, e_pad = _stage_a(x, enc1_w, enc1_b, enc2_w, enc2_b,
                            h0_w, h0_b, h1_w, h1_b, h2_w, h2_b,
                            jnp.asarray(B, f32))
    dec_w_p = jnp.pad(dec_w, ((0, 0), (0, _LP - _LAT))).astype(jnp.bfloat16)
    idx = jnp.pad(neighbour_id, ((0, _NPAD - _N), (0, 0))).reshape(-1)
    nd_t = jnp.pad(neighbour_distance, ((0, _NPAD - _N), (0, 0)))
    lab_f = jnp.pad(clustering_labels, (0, _NPAD - _N)).astype(f32)
    bias2 = jnp.pad(dec_b, (0, _NPAD - _N)).reshape(-1, 1)
    lab2 = lab_f.reshape(-1, 1)
    nq = 4
    qn = _NPAD // nq
    qr = _ROWS // nq
    outs = []
    for h in range(nq):
        g2 = _gather(dec_w_p, idx[h * qr:(h + 1) * qr]).reshape(qn, _KL)
        outs.append(_main(nd_t, g2, lab2, a_tab, e_pad, bias2,
                          h * (qn // _JB)))
    out_t = jnp.concatenate(outs, axis=0)
    return out_t[:_N, :].T


# speedup vs baseline: 1.0091x; 1.0091x over previous
import functools

import jax
import jax.numpy as jnp
from jax import lax
from jax.experimental import pallas as pl
from jax.experimental.pallas import tpu as pltpu
from jax.experimental.pallas import tpu_sc as plsc

_N = 10000
_NPAD = 10240
_K = 16
_LAT = 20
_LP = 32
_MU = 10.0
_BATCH = 16
_NCL = 50
_NB = 20
_JB = _NPAD // _NB
_KL = _K * _LP

_NW = 32
_ROWS = _NPAD * _K
_CHUNK = 640
_NBUF = 2


def _stage_a_body(x_ref, w1_ref, b1_ref, w2_ref, b2_ref, h0w_ref, h0b_ref,
                  h1w_ref, h1b_ref, h2w_ref, h2b_ref, bv_ref, a_ref, e_ref):
    f32 = jnp.float32
    hi = None
    pre = lax.dot_general(w1_ref[...].astype(jnp.bfloat16),
                          x_ref[...].astype(jnp.bfloat16),
                          (((1,), (1,)), ((), ())),
                          precision=hi,
                          preferred_element_type=jnp.float32
                          ) + b1_ref[...]
    s = jax.nn.sigmoid(pre)
    enc_t = lax.dot_general(w2_ref[...], s, (((1,), (0,)), ((), ())),
                            precision=hi) + b2_ref[...]
    h = lax.dot_general(h0w_ref[...], enc_t, (((1,), (0,)), ((), ())),
                        precision=hi) + h0b_ref[...]
    h = h * jax.nn.sigmoid(h)
    h = lax.dot_general(h1w_ref[...], h, (((1,), (0,)), ((), ())),
                        precision=hi) + h1b_ref[...]
    h = h * jax.nn.sigmoid(h)
    h = lax.dot_general(h2w_ref[...], h, (((1,), (0,)), ((), ())),
                        precision=hi) + h2b_ref[...]
    hot = jax.nn.sigmoid(0.01 * h)
    logu = jnp.log(1.0 - 0.5 * hot)
    c0 = (_MU * bv_ref[0, 0]) ** -2
    lvec = lax.broadcasted_iota(jnp.int32, (_NCL, _LP), 1).astype(f32)
    lmask = lvec < float(_LAT)
    for i in range(_BATCH):
        li = jnp.broadcast_to(logu[:, i:i + 1], (_NCL, _LP))
        a_ref[i] = jnp.where(lmask, c0 * jnp.exp(-2.0 * lvec * li), 0.0)
    e = jnp.transpose(enc_t)
    e_ref[...] = jnp.concatenate(
        [e, jnp.zeros((_BATCH, _LP - _LAT), f32)], axis=1)


def _stage_a(x, enc1_w, enc1_b, enc2_w, enc2_b, h0_w, h0_b, h1_w, h1_b,
             h2_w, h2_b, b_scalar):
    f32 = jnp.float32
    out_shape = (jax.ShapeDtypeStruct((_BATCH, _NCL, _LP), f32),
                 jax.ShapeDtypeStruct((_BATCH, _LP), f32))
    return pl.pallas_call(_stage_a_body, out_shape=out_shape)(
        x, enc1_w,
        enc1_b.reshape(-1, 1), enc2_w, enc2_b.reshape(-1, 1),
        h0_w, h0_b.reshape(-1, 1), h1_w, h1_b.reshape(-1, 1),
        h2_w, h2_b.reshape(-1, 1), b_scalar.reshape(1, 1))


def _gather(table, idx):
    n_rows = idx.shape[0]
    rpw = n_rows // _NW
    nch = rpw // _CHUNK
    crows = _CHUNK // _K
    mesh = plsc.VectorSubcoreMesh(core_axis_name="c", subcore_axis_name="s")

    @functools.partial(
        pl.kernel, mesh=mesh,
        compiler_params=pltpu.CompilerParams(use_tc_tiling_on_sc=False),
        out_type=jax.ShapeDtypeStruct((n_rows // _K, _KL), jnp.bfloat16),
        scratch_types=([pltpu.VMEM((rpw,), jnp.int32)]
                       + [pltpu.VMEM((_CHUNK, _LP), jnp.bfloat16)] * _NBUF
                       + [pltpu.VMEM((crows, _KL), jnp.bfloat16)] * _NBUF
                       + [pltpu.SemaphoreType.DMA] * (2 * _NBUF)),
    )
    def gk(table_hbm, idx_hbm, out_hbm, idx_v, *bufs_sems):
        rows = bufs_sems[:_NBUF]
        wide = bufs_sems[_NBUF:2 * _NBUF]
        gsem = bufs_sems[2 * _NBUF:3 * _NBUF]
        osem = bufs_sems[3 * _NBUF:]
        wid = lax.axis_index("s") * 2 + lax.axis_index("c")
        base = wid * rpw
        pltpu.sync_copy(idx_hbm.at[pl.ds(base, rpw)], idx_v)
        gh = {}
        for c in range(nch):
            b = c % _NBUF
            gh[c] = pltpu.async_copy(
                table_hbm.at[idx_v.at[pl.ds(c * _CHUNK, _CHUNK)]],
                rows[b], gsem[b])
        sh = {}
        for c in range(nch):
            b = c % _NBUF
            gh[c].wait()
            rv, wv = rows[b], wide[b]

            def pack(j, _, rv=rv, wv=wv):
                for s in range(_K):
                    wv[j, pl.ds(s * _LP, _LP)] = rv[j * _K + s]
                return 0

            lax.fori_loop(0, crows, pack, 0)
            sh[c] = pltpu.async_copy(
                wv, out_hbm.at[pl.ds(base // _K + c * crows, crows)],
                osem[b])
        for c in range(nch):
            sh[c].wait()

    return gk(table, idx)


def _main_body(nd_ref, g_ref, lab_ref, a_ref, e_ref, bias_ref, out_ref):
    f32 = jnp.float32
    bf16 = jnp.bfloat16
    nd = nd_ref[...]
    kt = ((lax.broadcasted_iota(jnp.int32, (_K, _KL), 1) // _LP) ==
          lax.broadcasted_iota(jnp.int32, (_K, _KL), 0)).astype(bf16)
    d2 = jnp.dot((nd * nd).astype(bf16), kt,
                 preferred_element_type=f32).astype(bf16)
    g = g_ref[...]
    onehot = (jnp.broadcast_to(lab_ref[...], (_JB, _NCL)) ==
              lax.broadcasted_iota(jnp.int32, (_JB, _NCL), 1)
              ).astype(bf16)
    seg = ((lax.broadcasted_iota(jnp.int32, (_KL, _LP), 0) % _LP) ==
           lax.broadcasted_iota(jnp.int32, (_KL, _LP), 1)).astype(bf16)
    a_bf = a_ref[...].astype(bf16)
    cols = []
    for i in range(_BATCH):
        a32 = jnp.dot(onehot, a_bf[i],
                      preferred_element_type=f32).astype(bf16)
        at = jnp.concatenate([a32] * _K, axis=1)
        r = jnp.maximum(1.0 - d2 * at, 0.0)
        rg = r * g
        zq = jnp.dot(jnp.concatenate([r, rg], axis=0), seg,
                     preferred_element_type=f32)
        z = zq[:_JB]
        q = zq[_JB:]
        ei = e_ref[i:i + 1, :]
        cols.append(jnp.sum(q * (ei / z), axis=1, keepdims=True))
    out_ref[...] = jnp.concatenate(cols, axis=1) + bias_ref[...]


def _main(nd_t, g2, lab_f, a_tab, e_pad, bias2, blk0):
    f32 = jnp.float32
    nblk = g2.shape[0] // _JB
    return pl.pallas_call(
        _main_body,
        grid=(nblk,),
        in_specs=[
            pl.BlockSpec((_JB, _K), lambda b: (b + blk0, 0)),
            pl.BlockSpec((_JB, _KL), lambda b: (b, 0)),
            pl.BlockSpec((_JB, 1), lambda b: (b + blk0, 0)),
            pl.BlockSpec((_BATCH, _NCL, _LP), lambda b: (0, 0, 0)),
            pl.BlockSpec((_BATCH, _LP), lambda b: (0, 0)),
            pl.BlockSpec((_JB, 1), lambda b: (b + blk0, 0)),
        ],
        out_specs=pl.BlockSpec((_JB, _BATCH), lambda b: (b, 0)),
        out_shape=jax.ShapeDtypeStruct((g2.shape[0], _BATCH), f32),
    )(nd_t, g2, lab_f, a_tab, e_pad, bias2)


def kernel(x, enc1_w, enc1_b, enc2_w, enc2_b, dec_w, dec_b, h0_w, h0_b,
           h1_w, h1_b, h2_w, h2_b, B, neighbour_id, neighbour_distance,
           clustering_labels):
    f32 = jnp.float32
    a_tab, e_pad = _stage_a(x, enc1_w, enc1_b, enc2_w, enc2_b,
                            h0_w, h0_b, h1_w, h1_b, h2_w, h2_b,
                            jnp.asarray(B, f32))
    dec_w_p = jnp.pad(dec_w, ((0, 0), (0, _LP - _LAT))).astype(jnp.bfloat16)
    idx = jnp.pad(neighbour_id, ((0, _NPAD - _N), (0, 0))).reshape(-1)
    nd_t = neighbour_distance
    lab2 = clustering_labels.reshape(-1, 1)
    bias2 = dec_b.reshape(-1, 1)
    nq = 4
    qn = _NPAD // nq
    qr = _ROWS // nq
    outs = []
    for h in range(nq):
        g2 = _gather(dec_w_p, idx[h * qr:(h + 1) * qr])
        outs.append(_main(nd_t, g2, lab2, a_tab, e_pad, bias2,
                          h * (qn // _JB)))
    out_t = jnp.concatenate(outs, axis=0)
    return out_t[:_N, :].T


# speedup vs baseline: 1.0443x; 1.0348x over previous
import functools

import jax
import jax.numpy as jnp
from jax import lax
from jax.experimental import pallas as pl
from jax.experimental.pallas import tpu as pltpu
from jax.experimental.pallas import tpu_sc as plsc

_N = 10000
_NPAD = 10240
_K = 16
_LAT = 20
_LP = 32
_MU = 10.0
_BATCH = 16
_NCL = 50
_NB = 20
_JB = _NPAD // _NB
_KL = _K * _LP

_NW = 32
_ROWS = _NPAD * _K
_CHUNK = 640
_NBUF = 2


def _stage_a_body(x_ref, w1_ref, b1_ref, w2_ref, b2_ref, h0w_ref, h0b_ref,
                  h1w_ref, h1b_ref, h2w_ref, h2b_ref, bv_ref, a_ref, e_ref):
    f32 = jnp.float32
    hi = None
    pre = lax.dot_general(w1_ref[...].astype(jnp.bfloat16),
                          x_ref[...].astype(jnp.bfloat16),
                          (((1,), (1,)), ((), ())),
                          precision=hi,
                          preferred_element_type=jnp.float32
                          ) + b1_ref[...]
    s = jax.nn.sigmoid(pre)
    enc_t = lax.dot_general(w2_ref[...], s, (((1,), (0,)), ((), ())),
                            precision=hi) + b2_ref[...]
    h = lax.dot_general(h0w_ref[...], enc_t, (((1,), (0,)), ((), ())),
                        precision=hi) + h0b_ref[...]
    h = h * jax.nn.sigmoid(h)
    h = lax.dot_general(h1w_ref[...], h, (((1,), (0,)), ((), ())),
                        precision=hi) + h1b_ref[...]
    h = h * jax.nn.sigmoid(h)
    h = lax.dot_general(h2w_ref[...], h, (((1,), (0,)), ((), ())),
                        precision=hi) + h2b_ref[...]
    hot = jax.nn.sigmoid(0.01 * h)
    logu = jnp.log(1.0 - 0.5 * hot)
    c0 = (_MU * bv_ref[0, 0]) ** -2
    lvec = lax.broadcasted_iota(jnp.int32, (_NCL, _LP), 1).astype(f32)
    lmask = lvec < float(_LAT)
    for i in range(_BATCH):
        li = jnp.broadcast_to(logu[:, i:i + 1], (_NCL, _LP))
        a_ref[i] = jnp.where(lmask, c0 * jnp.exp(-2.0 * lvec * li), 0.0)
    e = jnp.transpose(enc_t)
    e_ref[...] = jnp.concatenate(
        [e, jnp.zeros((_BATCH, _LP - _LAT), f32)], axis=1)


def _stage_a(x, enc1_w, enc1_b, enc2_w, enc2_b, h0_w, h0_b, h1_w, h1_b,
             h2_w, h2_b, b_scalar):
    f32 = jnp.float32
    out_shape = (jax.ShapeDtypeStruct((_BATCH, _NCL, _LP), f32),
                 jax.ShapeDtypeStruct((_BATCH, _LP), f32))
    return pl.pallas_call(_stage_a_body, out_shape=out_shape)(
        x, enc1_w,
        enc1_b.reshape(-1, 1), enc2_w, enc2_b.reshape(-1, 1),
        h0_w, h0_b.reshape(-1, 1), h1_w, h1_b.reshape(-1, 1),
        h2_w, h2_b.reshape(-1, 1), b_scalar.reshape(1, 1))


def _gather(table, idx):
    n_rows = idx.shape[0]
    rpw = n_rows // _NW
    nch = rpw // _CHUNK
    crows = _CHUNK // _K
    mesh = plsc.VectorSubcoreMesh(core_axis_name="c", subcore_axis_name="s")

    @functools.partial(
        pl.kernel, mesh=mesh,
        compiler_params=pltpu.CompilerParams(use_tc_tiling_on_sc=False),
        out_type=jax.ShapeDtypeStruct((n_rows // _K, _KL), jnp.bfloat16),
        scratch_types=([pltpu.VMEM((rpw,), jnp.int32)]
                       + [pltpu.VMEM((_CHUNK, _LP), jnp.bfloat16)] * _NBUF
                       + [pltpu.VMEM((crows, _KL), jnp.bfloat16)] * _NBUF
                       + [pltpu.SemaphoreType.DMA] * (2 * _NBUF)),
    )
    def gk(table_hbm, idx_hbm, out_hbm, idx_v, *bufs_sems):
        rows = bufs_sems[:_NBUF]
        wide = bufs_sems[_NBUF:2 * _NBUF]
        gsem = bufs_sems[2 * _NBUF:3 * _NBUF]
        osem = bufs_sems[3 * _NBUF:]
        wid = lax.axis_index("s") * 2 + lax.axis_index("c")
        base = wid * rpw
        pltpu.sync_copy(idx_hbm.at[pl.ds(base, rpw)], idx_v)
        gh = {}
        for c in range(nch):
            b = c % _NBUF
            gh[c] = pltpu.async_copy(
                table_hbm.at[idx_v.at[pl.ds(c * _CHUNK, _CHUNK)]],
                rows[b], gsem[b])
        sh = {}
        for c in range(nch):
            b = c % _NBUF
            gh[c].wait()
            rv, wv = rows[b], wide[b]

            def pack(j, _, rv=rv, wv=wv):
                for s in range(_K):
                    wv[j, pl.ds(s * _LP, _LP)] = rv[j * _K + s]
                return 0

            lax.fori_loop(0, crows, pack, 0)
            sh[c] = pltpu.async_copy(
                wv, out_hbm.at[pl.ds(base // _K + c * crows, crows)],
                osem[b])
        for c in range(nch):
            sh[c].wait()

    return gk(table, idx)


def _main_body(nlb_ref, g_hbm, a_ref, e_ref, out_ref, gbuf, gsem):
    f32 = jnp.float32
    bf16 = jnp.bfloat16
    b = pl.program_id(0)
    nblk = pl.num_programs(0)
    slot = lax.rem(b, 2)
    nxt = lax.rem(b + 1, 2)

    @pl.when(b == 0)
    def _():
        pltpu.make_async_copy(g_hbm.at[pl.ds(0, _JB)], gbuf.at[0],
                              gsem.at[0]).start()

    @pl.when(b + 1 < nblk)
    def _():
        pltpu.make_async_copy(g_hbm.at[pl.ds((b + 1) * _JB, _JB)],
                              gbuf.at[nxt], gsem.at[nxt]).start()

    nlb = nlb_ref[...]
    nd = nlb[:, :_K]
    lab = nlb[:, _K:_K + 1]
    bias = nlb[:, _K + 1:]
    kt = ((lax.broadcasted_iota(jnp.int32, (_K, _KL), 1) // _LP) ==
          lax.broadcasted_iota(jnp.int32, (_K, _KL), 0)).astype(bf16)
    d2 = jnp.dot((nd * nd).astype(bf16), kt,
                 preferred_element_type=f32).astype(bf16)
    pltpu.make_async_copy(g_hbm.at[pl.ds(b * _JB, _JB)], gbuf.at[slot],
                          gsem.at[slot]).wait()
    g = gbuf[slot]
    onehot = (jnp.broadcast_to(lab, (_JB, _NCL)) ==
              lax.broadcasted_iota(jnp.int32, (_JB, _NCL), 1).astype(f32)
              ).astype(bf16)
    seg = ((lax.broadcasted_iota(jnp.int32, (_KL, _LP), 0) % _LP) ==
           lax.broadcasted_iota(jnp.int32, (_KL, _LP), 1)).astype(bf16)
    a_bf = a_ref[...].astype(bf16)
    cols = []
    for i in range(_BATCH):
        a32 = jnp.dot(onehot, a_bf[i],
                      preferred_element_type=f32).astype(bf16)
        at = jnp.concatenate([a32] * _K, axis=1)
        r = jnp.maximum(1.0 - d2 * at, 0.0)
        rg = r * g
        zq = jnp.dot(jnp.concatenate([r, rg], axis=0), seg,
                     preferred_element_type=f32)
        z = zq[:_JB]
        q = zq[_JB:]
        ei = e_ref[i:i + 1, :]
        cols.append(jnp.sum(q * (ei / z), axis=1, keepdims=True))
    out_ref[...] = jnp.concatenate(cols, axis=1) + bias


def _main(nlb, g2, a_tab, e_pad, blk0):
    f32 = jnp.float32
    nblk = g2.shape[0] // _JB
    return pl.pallas_call(
        _main_body,
        grid=(nblk,),
        in_specs=[
            pl.BlockSpec((_JB, 18), lambda b: (b + blk0, 0)),
            pl.BlockSpec(memory_space=pl.ANY),
            pl.BlockSpec((_BATCH, _NCL, _LP), lambda b: (0, 0, 0)),
            pl.BlockSpec((_BATCH, _LP), lambda b: (0, 0)),
        ],
        out_specs=pl.BlockSpec((_JB, _BATCH), lambda b: (b, 0)),
        out_shape=jax.ShapeDtypeStruct((g2.shape[0], _BATCH), f32),
        scratch_shapes=[pltpu.VMEM((2, _JB, _KL), jnp.bfloat16),
                        pltpu.SemaphoreType.DMA((2,))],
    )(nlb, g2, a_tab, e_pad)


def kernel(x, enc1_w, enc1_b, enc2_w, enc2_b, dec_w, dec_b, h0_w, h0_b,
           h1_w, h1_b, h2_w, h2_b, B, neighbour_id, neighbour_distance,
           clustering_labels):
    f32 = jnp.float32
    a_tab, e_pad = _stage_a(x, enc1_w, enc1_b, enc2_w, enc2_b,
                            h0_w, h0_b, h1_w, h1_b, h2_w, h2_b,
                            jnp.asarray(B, f32))
    dec_w_p = jnp.pad(dec_w, ((0, 0), (0, _LP - _LAT))).astype(jnp.bfloat16)
    idx = jnp.pad(neighbour_id, ((0, _NPAD - _N), (0, 0))).reshape(-1)
    nlb = jnp.concatenate(
        [neighbour_distance, clustering_labels.astype(f32)[:, None],
         dec_b[:, None]], axis=1)
    nq = 4
    qn = _NPAD // nq
    qr = _ROWS // nq
    outs = []
    for h in range(nq):
        g2 = _gather(dec_w_p, idx[h * qr:(h + 1) * qr])
        outs.append(_main(nlb, g2, a_tab, e_pad, h * (qn // _JB)))
    out_t = jnp.concatenate(outs, axis=0)
    return out_t[:_N, :].T


# speedup vs baseline: 1.0453x; 1.0010x over previous
import functools

import jax
import jax.numpy as jnp
from jax import lax
from jax.experimental import pallas as pl
from jax.experimental.pallas import tpu as pltpu
from jax.experimental.pallas import tpu_sc as plsc

_N = 10000
_NPAD = 10240
_K = 16
_LAT = 20
_LP = 32
_MU = 10.0
_BATCH = 16
_NCL = 50
_NB = 20
_JB = _NPAD // _NB
_KL = _K * _LP

_NW = 32
_ROWS = _NPAD * _K
_CHUNK = 640
_NBUF = 2


def _stage_a_body(x_ref, w1_ref, b1_ref, w2_ref, b2_ref, h0w_ref, h0b_ref,
                  h1w_ref, h1b_ref, h2w_ref, h2b_ref, bv_ref, a_ref, e_ref):
    f32 = jnp.float32
    hi = None
    pre = lax.dot_general(w1_ref[...].astype(jnp.bfloat16),
                          x_ref[...].astype(jnp.bfloat16),
                          (((1,), (1,)), ((), ())),
                          precision=hi,
                          preferred_element_type=jnp.float32
                          ) + b1_ref[...]
    s = jax.nn.sigmoid(pre)
    enc_t = lax.dot_general(w2_ref[...], s, (((1,), (0,)), ((), ())),
                            precision=hi) + b2_ref[...]
    h = lax.dot_general(h0w_ref[...], enc_t, (((1,), (0,)), ((), ())),
                        precision=hi) + h0b_ref[...]
    h = h * jax.nn.sigmoid(h)
    h = lax.dot_general(h1w_ref[...], h, (((1,), (0,)), ((), ())),
                        precision=hi) + h1b_ref[...]
    h = h * jax.nn.sigmoid(h)
    h = lax.dot_general(h2w_ref[...], h, (((1,), (0,)), ((), ())),
                        precision=hi) + h2b_ref[...]
    hot = jax.nn.sigmoid(0.01 * h)
    logu = jnp.log(1.0 - 0.5 * hot)
    c0 = (_MU * bv_ref[0, 0]) ** -2
    lvec = lax.broadcasted_iota(jnp.int32, (_NCL, _LP), 1).astype(f32)
    lmask = lvec < float(_LAT)
    for i in range(_BATCH):
        li = jnp.broadcast_to(logu[:, i:i + 1], (_NCL, _LP))
        a_ref[i] = jnp.where(lmask, c0 * jnp.exp(-2.0 * lvec * li), 0.0)
    e = jnp.transpose(enc_t)
    e_ref[...] = jnp.concatenate(
        [e, jnp.zeros((_BATCH, _LP - _LAT), f32)], axis=1)


def _stage_a(x, enc1_w, enc1_b, enc2_w, enc2_b, h0_w, h0_b, h1_w, h1_b,
             h2_w, h2_b, b_scalar):
    f32 = jnp.float32
    out_shape = (jax.ShapeDtypeStruct((_BATCH, _NCL, _LP), f32),
                 jax.ShapeDtypeStruct((_BATCH, _LP), f32))
    return pl.pallas_call(_stage_a_body, out_shape=out_shape)(
        x, enc1_w,
        enc1_b.reshape(-1, 1), enc2_w, enc2_b.reshape(-1, 1),
        h0_w, h0_b.reshape(-1, 1), h1_w, h1_b.reshape(-1, 1),
        h2_w, h2_b.reshape(-1, 1), b_scalar.reshape(1, 1))


def _gather(table, idx):
    n_rows = idx.shape[0]
    rpw = n_rows // _NW
    nch = rpw // _CHUNK
    crows = _CHUNK // _K
    mesh = plsc.VectorSubcoreMesh(core_axis_name="c", subcore_axis_name="s")

    @functools.partial(
        pl.kernel, mesh=mesh,
        compiler_params=pltpu.CompilerParams(use_tc_tiling_on_sc=False),
        out_type=jax.ShapeDtypeStruct((n_rows // _K, _KL), jnp.bfloat16),
        scratch_types=([pltpu.VMEM((rpw,), jnp.int32)]
                       + [pltpu.VMEM((_CHUNK, _LP), jnp.bfloat16)] * _NBUF
                       + [pltpu.VMEM((crows, _KL), jnp.bfloat16)] * _NBUF
                       + [pltpu.SemaphoreType.DMA] * (2 * _NBUF)),
    )
    def gk(table_hbm, idx_hbm, out_hbm, idx_v, *bufs_sems):
        rows = bufs_sems[:_NBUF]
        wide = bufs_sems[_NBUF:2 * _NBUF]
        gsem = bufs_sems[2 * _NBUF:3 * _NBUF]
        osem = bufs_sems[3 * _NBUF:]
        wid = lax.axis_index("s") * 2 + lax.axis_index("c")
        base = wid * rpw
        pltpu.sync_copy(idx_hbm.at[pl.ds(base, rpw)], idx_v)
        gh = {}
        for c in range(nch):
            b = c % _NBUF
            gh[c] = pltpu.async_copy(
                table_hbm.at[idx_v.at[pl.ds(c * _CHUNK, _CHUNK)]],
                rows[b], gsem[b])
        sh = {}
        for c in range(nch):
            b = c % _NBUF
            gh[c].wait()
            rv, wv = rows[b], wide[b]

            def pack(j, _, rv=rv, wv=wv):
                for s in range(_K):
                    wv[j, pl.ds(s * _LP, _LP)] = rv[j * _K + s]
                return 0

            lax.fori_loop(0, crows, pack, 0)
            sh[c] = pltpu.async_copy(
                wv, out_hbm.at[pl.ds(base // _K + c * crows, crows)],
                osem[b])
        for c in range(nch):
            sh[c].wait()

    return gk(table, idx)


def _main_body(nlb_ref, g_hbm, a_ref, e_ref, out_ref, gbuf, gsem):
    f32 = jnp.float32
    bf16 = jnp.bfloat16
    b = pl.program_id(0)
    nblk = pl.num_programs(0)
    slot = lax.rem(b, 2)
    nxt = lax.rem(b + 1, 2)

    @pl.when(b == 0)
    def _():
        pltpu.make_async_copy(g_hbm.at[pl.ds(0, _JB)], gbuf.at[0],
                              gsem.at[0]).start()

    @pl.when(b + 1 < nblk)
    def _():
        pltpu.make_async_copy(g_hbm.at[pl.ds((b + 1) * _JB, _JB)],
                              gbuf.at[nxt], gsem.at[nxt]).start()

    nlb = nlb_ref[...]
    nd = nlb[:, :_K]
    lab = nlb[:, _K:_K + 1]
    bias = nlb[:, _K + 1:]
    kt = ((lax.broadcasted_iota(jnp.int32, (_K, _KL), 1) // _LP) ==
          lax.broadcasted_iota(jnp.int32, (_K, _KL), 0)).astype(bf16)
    d2 = jnp.dot((nd * nd).astype(bf16), kt,
                 preferred_element_type=f32).astype(bf16)
    pltpu.make_async_copy(g_hbm.at[pl.ds(b * _JB, _JB)], gbuf.at[slot],
                          gsem.at[slot]).wait()
    g = gbuf[slot]
    onehot = (jnp.broadcast_to(lab, (_JB, _NCL)) ==
              lax.broadcasted_iota(jnp.int32, (_JB, _NCL), 1).astype(f32)
              ).astype(bf16)
    seg = ((lax.broadcasted_iota(jnp.int32, (_KL, _LP), 0) % _LP) ==
           lax.broadcasted_iota(jnp.int32, (_KL, _LP), 1)).astype(bf16)
    a_bf = a_ref[...].astype(bf16)
    cols = []
    for i in range(_BATCH):
        a32 = jnp.dot(onehot, a_bf[i],
                      preferred_element_type=f32).astype(bf16)
        at = jnp.concatenate([a32] * _K, axis=1)
        r = jnp.maximum(1.0 - d2 * at, 0.0)
        rg = r * g
        zq = jnp.dot(jnp.concatenate([r, rg], axis=0), seg,
                     preferred_element_type=f32)
        z = zq[:_JB]
        q = zq[_JB:]
        ei = e_ref[i:i + 1, :]
        cols.append(jnp.sum(q * (ei / z), axis=1, keepdims=True))
    out_ref[...] = jnp.concatenate(cols, axis=1) + bias


def _main(nlb, g2, a_tab, e_pad, blk0):
    f32 = jnp.float32
    nblk = g2.shape[0] // _JB
    return pl.pallas_call(
        _main_body,
        grid=(nblk,),
        in_specs=[
            pl.BlockSpec((_JB, 18), lambda b: (b + blk0, 0)),
            pl.BlockSpec(memory_space=pl.ANY),
            pl.BlockSpec((_BATCH, _NCL, _LP), lambda b: (0, 0, 0)),
            pl.BlockSpec((_BATCH, _LP), lambda b: (0, 0)),
        ],
        out_specs=pl.BlockSpec((_JB, _BATCH), lambda b: (b, 0)),
        out_shape=jax.ShapeDtypeStruct((g2.shape[0], _BATCH), f32),
        scratch_shapes=[pltpu.VMEM((2, _JB, _KL), jnp.bfloat16),
                        pltpu.SemaphoreType.DMA((2,))],
    )(nlb, g2, a_tab, e_pad)


def kernel(x, enc1_w, enc1_b, enc2_w, enc2_b, dec_w, dec_b, h0_w, h0_b,
           h1_w, h1_b, h2_w, h2_b, B, neighbour_id, neighbour_distance,
           clustering_labels):
    f32 = jnp.float32
    a_tab, e_pad = _stage_a(x, enc1_w, enc1_b, enc2_w, enc2_b,
                            h0_w, h0_b, h1_w, h1_b, h2_w, h2_b,
                            jnp.asarray(B, f32))
    dec_w_p = jnp.pad(dec_w, ((0, 0), (0, _LP - _LAT))).astype(jnp.bfloat16)
    idx = jnp.pad(neighbour_id, ((0, _NPAD - _N), (0, 0))).reshape(-1)
    nlb = jnp.concatenate(
        [neighbour_distance, clustering_labels.astype(f32)[:, None],
         dec_b[:, None]], axis=1)
    nq = 4
    qn = _NPAD // nq
    qr = _ROWS // nq
    g2s = [_gather(dec_w_p, idx[h * qr:(h + 1) * qr]) for h in range(nq)]
    outs = [_main(nlb, g2s[h], a_tab, e_pad, h * (qn // _JB))
            for h in range(nq)]
    out_t = jnp.concatenate(outs, axis=0)
    return out_t[:_N, :].T


# speedup vs baseline: 1.1371x; 1.0878x over previous
import functools

import jax
import jax.numpy as jnp
from jax import lax
from jax.experimental import pallas as pl
from jax.experimental.pallas import tpu as pltpu
from jax.experimental.pallas import tpu_sc as plsc

_N = 10000
_NPAD = 10240
_K = 16
_LAT = 20
_LP = 32
_MU = 10.0
_BATCH = 16
_NCL = 50
_JB = 1280
_KL = _K * _LP

_NW = 32
_ROWS = _NPAD * _K
_CHUNK = 640
_NBUF = 2


def _stage_a_body(x_ref, w1_ref, b1_ref, w2_ref, b2_ref, h0w_ref, h0b_ref,
                  h1w_ref, h1b_ref, h2w_ref, h2b_ref, bv_ref, a_ref, e_ref):
    f32 = jnp.float32
    hi = None
    pre = lax.dot_general(w1_ref[...].astype(jnp.bfloat16),
                          x_ref[...].astype(jnp.bfloat16),
                          (((1,), (1,)), ((), ())),
                          precision=hi,
                          preferred_element_type=jnp.float32
                          ) + b1_ref[...]
    s = jax.nn.sigmoid(pre)
    enc_t = lax.dot_general(w2_ref[...], s, (((1,), (0,)), ((), ())),
                            precision=hi) + b2_ref[...]
    h = lax.dot_general(h0w_ref[...], enc_t, (((1,), (0,)), ((), ())),
                        precision=hi) + h0b_ref[...]
    h = h * jax.nn.sigmoid(h)
    h = lax.dot_general(h1w_ref[...], h, (((1,), (0,)), ((), ())),
                        precision=hi) + h1b_ref[...]
    h = h * jax.nn.sigmoid(h)
    h = lax.dot_general(h2w_ref[...], h, (((1,), (0,)), ((), ())),
                        precision=hi) + h2b_ref[...]
    hot = jax.nn.sigmoid(0.01 * h)
    logu = jnp.log(1.0 - 0.5 * hot)
    c0 = (_MU * bv_ref[0, 0]) ** -2
    lvec = lax.broadcasted_iota(jnp.int32, (_NCL, _LP), 1).astype(f32)
    lmask = lvec < float(_LAT)
    for i in range(_BATCH):
        li = jnp.broadcast_to(logu[:, i:i + 1], (_NCL, _LP))
        a_ref[i] = jnp.where(lmask, c0 * jnp.exp(-2.0 * lvec * li), 0.0)
    e = jnp.transpose(enc_t)
    e_ref[...] = jnp.concatenate(
        [e, jnp.zeros((_BATCH, _LP - _LAT), f32)], axis=1)


def _stage_a(x, enc1_w, enc1_b, enc2_w, enc2_b, h0_w, h0_b, h1_w, h1_b,
             h2_w, h2_b, b_scalar):
    f32 = jnp.float32
    out_shape = (jax.ShapeDtypeStruct((_BATCH, _NCL, _LP), f32),
                 jax.ShapeDtypeStruct((_BATCH, _LP), f32))
    return pl.pallas_call(_stage_a_body, out_shape=out_shape)(
        x, enc1_w,
        enc1_b.reshape(-1, 1), enc2_w, enc2_b.reshape(-1, 1),
        h0_w, h0_b.reshape(-1, 1), h1_w, h1_b.reshape(-1, 1),
        h2_w, h2_b.reshape(-1, 1), b_scalar.reshape(1, 1))


def _gather(table, idx):
    n_rows = idx.shape[0]
    rpw = n_rows // _NW
    nch = rpw // _CHUNK
    crows = _CHUNK // _K
    mesh = plsc.VectorSubcoreMesh(core_axis_name="c", subcore_axis_name="s")

    @functools.partial(
        pl.kernel, mesh=mesh,
        compiler_params=pltpu.CompilerParams(use_tc_tiling_on_sc=False),
        out_type=jax.ShapeDtypeStruct((n_rows // _K, _KL), jnp.bfloat16),
        scratch_types=([pltpu.VMEM((rpw,), jnp.int32)]
                       + [pltpu.VMEM((_CHUNK, _LP), jnp.bfloat16)] * _NBUF
                       + [pltpu.VMEM((crows, _KL), jnp.bfloat16)] * _NBUF
                       + [pltpu.SemaphoreType.DMA] * (2 * _NBUF)),
    )
    def gk(table_hbm, idx_hbm, out_hbm, idx_v, *bufs_sems):
        rows = bufs_sems[:_NBUF]
        wide = bufs_sems[_NBUF:2 * _NBUF]
        gsem = bufs_sems[2 * _NBUF:3 * _NBUF]
        osem = bufs_sems[3 * _NBUF:]
        wid = lax.axis_index("s") * 2 + lax.axis_index("c")
        base = wid * rpw
        pltpu.sync_copy(idx_hbm.at[pl.ds(base, rpw)], idx_v)
        gh = {}
        for c in range(nch):
            b = c % _NBUF
            gh[c] = pltpu.async_copy(
                table_hbm.at[idx_v.at[pl.ds(c * _CHUNK, _CHUNK)]],
                rows[b], gsem[b])
        sh = {}
        for c in range(nch):
            b = c % _NBUF
            gh[c].wait()
            rv, wv = rows[b], wide[b]

            def pack(j, _, rv=rv, wv=wv):
                for s in range(_K):
                    wv[j, pl.ds(s * _LP, _LP)] = rv[j * _K + s]
                return 0

            lax.fori_loop(0, crows, pack, 0)
            sh[c] = pltpu.async_copy(
                wv, out_hbm.at[pl.ds(base // _K + c * crows, crows)],
                osem[b])
        for c in range(nch):
            sh[c].wait()

    return gk(table, idx)


def _main_body(nlb_ref, g_hbm, a_ref, e_ref, out_ref, gbuf, gsem):
    f32 = jnp.float32
    bf16 = jnp.bfloat16
    b = pl.program_id(0)
    nblk = pl.num_programs(0)
    slot = lax.rem(b, 2)
    nxt = lax.rem(b + 1, 2)

    @pl.when(b == 0)
    def _():
        pltpu.make_async_copy(g_hbm.at[pl.ds(0, _JB)], gbuf.at[0],
                              gsem.at[0]).start()

    @pl.when(b + 1 < nblk)
    def _():
        pltpu.make_async_copy(g_hbm.at[pl.ds((b + 1) * _JB, _JB)],
                              gbuf.at[nxt], gsem.at[nxt]).start()

    nlb = nlb_ref[...]
    nd = nlb[:, :_K]
    lab = nlb[:, _K:_K + 1]
    bias = nlb[:, _K + 1:]
    kt = ((lax.broadcasted_iota(jnp.int32, (_K, _KL), 1) // _LP) ==
          lax.broadcasted_iota(jnp.int32, (_K, _KL), 0)).astype(bf16)
    d2 = jnp.dot((nd * nd).astype(bf16), kt,
                 preferred_element_type=f32).astype(bf16)
    pltpu.make_async_copy(g_hbm.at[pl.ds(b * _JB, _JB)], gbuf.at[slot],
                          gsem.at[slot]).wait()
    g = gbuf[slot]
    onehot = (jnp.broadcast_to(lab, (_JB, _NCL)) ==
              lax.broadcasted_iota(jnp.int32, (_JB, _NCL), 1).astype(f32)
              ).astype(bf16)
    seg = ((lax.broadcasted_iota(jnp.int32, (_KL, _LP), 0) % _LP) ==
           lax.broadcasted_iota(jnp.int32, (_KL, _LP), 1)).astype(bf16)
    a_bf = a_ref[...].astype(bf16)
    cols = []
    for i in range(_BATCH):
        a32 = jnp.dot(onehot, a_bf[i],
                      preferred_element_type=f32).astype(bf16)
        at = jnp.concatenate([a32] * _K, axis=1)
        r = jnp.maximum(1.0 - d2 * at, 0.0)
        rg = r * g
        zq = jnp.dot(jnp.concatenate([r, rg], axis=0), seg,
                     preferred_element_type=f32)
        z = zq[:_JB]
        q = zq[_JB:]
        ei = e_ref[i:i + 1, :]
        cols.append(jnp.sum(q * (ei / z), axis=1, keepdims=True))
    out_ref[...] = jnp.concatenate(cols, axis=1) + bias


def _main(nlb, g2, a_tab, e_pad, blk0):
    f32 = jnp.float32
    nblk = g2.shape[0] // _JB
    return pl.pallas_call(
        _main_body,
        grid=(nblk,),
        in_specs=[
            pl.BlockSpec((_JB, 18), lambda b: (b + blk0, 0)),
            pl.BlockSpec(memory_space=pl.ANY),
            pl.BlockSpec((_BATCH, _NCL, _LP), lambda b: (0, 0, 0)),
            pl.BlockSpec((_BATCH, _LP), lambda b: (0, 0)),
        ],
        out_specs=pl.BlockSpec((_JB, _BATCH), lambda b: (b, 0)),
        out_shape=jax.ShapeDtypeStruct((g2.shape[0], _BATCH), f32),
        scratch_shapes=[pltpu.VMEM((2, _JB, _KL), jnp.bfloat16),
                        pltpu.SemaphoreType.DMA((2,))],
    )(nlb, g2, a_tab, e_pad)


def kernel(x, enc1_w, enc1_b, enc2_w, enc2_b, dec_w, dec_b, h0_w, h0_b,
           h1_w, h1_b, h2_w, h2_b, B, neighbour_id, neighbour_distance,
           clustering_labels):
    f32 = jnp.float32
    a_tab, e_pad = _stage_a(x, enc1_w, enc1_b, enc2_w, enc2_b,
                            h0_w, h0_b, h1_w, h1_b, h2_w, h2_b,
                            jnp.asarray(B, f32))
    dec_w_p = jnp.pad(dec_w, ((0, 0), (0, _LP - _LAT))).astype(jnp.bfloat16)
    idx = jnp.pad(neighbour_id, ((0, _NPAD - _N), (0, 0))).reshape(-1)
    nlb = jnp.concatenate(
        [neighbour_distance, clustering_labels.astype(f32)[:, None],
         dec_b[:, None]], axis=1)
    nq = 4
    qn = _NPAD // nq
    qr = _ROWS // nq
    g2s = [_gather(dec_w_p, idx[h * qr:(h + 1) * qr]) for h in range(nq)]
    outs = [_main(nlb, g2s[h], a_tab, e_pad, h * (qn // _JB))
            for h in range(nq)]
    out_t = jnp.concatenate(outs, axis=0)
    return out_t[:_N, :].T
